# baseline (speedup 1.0000x reference)
import jax
import jax.numpy as jnp
from jax import lax
from jax.experimental import pallas as pl
from jax.experimental.pallas import tpu as pltpu
from jax.experimental.pallas import tpu_sc as plsc

NTIME = 2048
NBATCH = 4096
LANES = 16
NWORKERS = 32
W = NBATCH // NWORKERS
NG = W // LANES
CSTRIDE = 16
NC_ROWS = NTIME // CSTRIDE


def _interp_body(times_hbm, values_hbm, t_hbm, out_hbm,
                 t_v, idx_v, idx2_v, idx3_v, coarse_v, fine_v, finv_v,
                 finv0_v, finv1_v, ttail_v, vtail_v, out_v, sem, sem2):
    nc = 2
    wid = lax.axis_index("s") * nc + lax.axis_index("c")
    base = wid * W
    cs = pl.ds(base, W)

    lane = lax.iota(jnp.int32, LANES)

    for g in range(NG):
        idx_v[pl.ds(g * LANES, LANES)] = (lane + g * LANES) * CSTRIDE + (
            CSTRIDE - 1)
    ct = pltpu.async_copy(times_hbm.at[idx_v, cs], coarse_v, sem)
    tt = pltpu.async_copy(times_hbm.at[pl.ds(NTIME - 8, 8), cs], ttail_v,
                          sem2)
    vt = pltpu.async_copy(values_hbm.at[pl.ds(NTIME - 8, 8), cs], vtail_v,
                          sem2)
    pltpu.sync_copy(t_hbm.at[cs], t_v)
    t_regs = [t_v[pl.ds(g * LANES, LANES)] for g in range(NG)]
    ct.wait()

    loc = [lane + g * LANES for g in range(NG)]
    posc = [jnp.zeros((LANES,), jnp.int32) for _ in range(NG)]
    zero = jnp.zeros((LANES,), jnp.float32)
    tk = [zero for _ in range(NG)]
    tk1 = [zero for _ in range(NG)]
    step = NC_ROWS
    while step >= 1:
        for g in range(NG):
            row = jnp.minimum(posc[g] + (step - 1), NC_ROWS - 1)
            val = plsc.load_gather(coarse_v, [row, loc[g]])
            valid = posc[g] + step <= NC_ROWS
            le = val <= t_regs[g]
            ok = jnp.logical_and(valid, le)
            fail = jnp.logical_and(valid, jnp.logical_not(le))
            tk[g] = jnp.where(ok, val, tk[g])
            tk1[g] = jnp.where(fail, val, tk1[g])
            posc[g] = posc[g] + jnp.where(ok, step, 0)
        step //= 2

    pos = [p * CSTRIDE for p in posc]
    step = CSTRIDE // 2
    while step >= 2:
        for g in range(NG):
            row = jnp.minimum(pos[g] + (step - 1), NTIME - 1)
            idx_v[pl.ds(g * LANES, LANES)] = row
        pltpu.async_copy(times_hbm.at[idx_v, cs], fine_v, sem).wait()
        for g in range(NG):
            val = plsc.load_gather(fine_v, [loc[g], loc[g]])
            valid = pos[g] + step <= NTIME
            le = val <= t_regs[g]
            ok = jnp.logical_and(valid, le)
            fail = jnp.logical_and(valid, jnp.logical_not(le))
            tk[g] = jnp.where(ok, val, tk[g])
            tk1[g] = jnp.where(fail, val, tk1[g])
            pos[g] = pos[g] + jnp.where(ok, step, 0)
        step //= 2

    for g in range(NG):
        p = pos[g]
        idx_v[pl.ds(g * LANES, LANES)] = jnp.minimum(p, NTIME - 1)
        idx2_v[pl.ds(g * LANES, LANES)] = jnp.maximum(p - 1, 0)
        idx3_v[pl.ds(g * LANES, LANES)] = jnp.minimum(p + 1, NTIME - 1)
    cp = pltpu.async_copy(times_hbm.at[idx_v, cs], fine_v, sem)
    cm = pltpu.async_copy(values_hbm.at[idx2_v, cs], finv_v, sem)
    c0 = pltpu.async_copy(values_hbm.at[idx_v, cs], finv0_v, sem)
    c1 = pltpu.async_copy(values_hbm.at[idx3_v, cs], finv1_v, sem)
    tt.wait()
    vt.wait()
    cp.wait()
    cm.wait()
    c0.wait()
    c1.wait()

    for g in range(NG):
        val_p = plsc.load_gather(fine_v, [loc[g], loc[g]])
        v_m = plsc.load_gather(finv_v, [loc[g], loc[g]])
        v_0 = plsc.load_gather(finv0_v, [loc[g], loc[g]])
        v_1 = plsc.load_gather(finv1_v, [loc[g], loc[g]])
        gsl = pl.ds(g * LANES, LANES)
        valid = pos[g] + 1 <= NTIME
        le = val_p <= t_regs[g]
        ok = jnp.logical_and(valid, le)
        fail = jnp.logical_and(valid, jnp.logical_not(le))
        tkg = jnp.where(ok, val_p, tk[g])
        tk1g = jnp.where(fail, val_p, tk1[g])
        n = pos[g] + jnp.where(ok, 1, 0)
        gi = jnp.bitwise_and(n, NTIME - 1)
        wrap = gi == 0
        vk = jnp.where(ok, v_0, v_m)
        vk1 = jnp.where(ok, v_1, v_0)
        tkf = jnp.where(wrap, ttail_v[6, gsl], tkg)
        tk1f = jnp.where(wrap, ttail_v[7, gsl], tk1g)
        vkf = jnp.where(wrap, vtail_v[6, gsl], vk)
        vk1f = jnp.where(wrap, vtail_v[7, gsl], vk1)
        s0 = (vk1f - vkf) / (tk1f - tkf)
        v0 = jnp.where(wrap, vk1f, vkf)
        t0 = jnp.where(wrap, tk1f, tkf)
        out_v[gsl] = v0 + s0 * (t_regs[g] - t0)

    pltpu.sync_copy(out_v, out_hbm.at[cs])


def kernel(times, values, t):
    mesh = plsc.VectorSubcoreMesh(core_axis_name="c", subcore_axis_name="s")
    f = pl.kernel(
        _interp_body,
        mesh=mesh,
        out_type=jax.ShapeDtypeStruct((NBATCH,), jnp.float32),
        compiler_params=pltpu.CompilerParams(needs_layout_passes=False),
        scratch_types=[
            pltpu.VMEM((W,), jnp.float32),
            pltpu.VMEM((W,), jnp.int32),
            pltpu.VMEM((W,), jnp.int32),
            pltpu.VMEM((W,), jnp.int32),
            pltpu.VMEM((NC_ROWS, W), jnp.float32),
            pltpu.VMEM((W, W), jnp.float32),
            pltpu.VMEM((W, W), jnp.float32),
            pltpu.VMEM((W, W), jnp.float32),
            pltpu.VMEM((W, W), jnp.float32),
            pltpu.VMEM((8, W), jnp.float32),
            pltpu.VMEM((8, W), jnp.float32),
            pltpu.VMEM((W,), jnp.float32),
            pltpu.SemaphoreType.DMA,
            pltpu.SemaphoreType.DMA,
        ],
    )
    return f(times, values, t)
